# baseline (speedup 1.0000x reference)
import functools

import jax
import jax.numpy as jnp
from jax import lax
from jax.experimental import pallas as pl
from jax.experimental.pallas import tpu as pltpu
from jax.experimental.pallas import tpu_sc as plsc

_L = 16


def _pos_embed_sc(row_embed, col_embed, B, H, W, D):
    C = 2 * D
    NW = 32
    assert H == NW and W <= col_embed.shape[0]
    NROW, DROW = row_embed.shape
    mesh = plsc.VectorSubcoreMesh(core_axis_name="c", subcore_axis_name="s")

    @functools.partial(
        pl.kernel,
        mesh=mesh,
        out_type=jax.ShapeDtypeStruct((B, H, W, C), jnp.float32),
        scratch_types=[
            pltpu.VMEM((1, DROW), jnp.float32),
            pltpu.VMEM((1, W, C), jnp.float32),
            pltpu.SemaphoreType.DMA,
            pltpu.SemaphoreType.DMA,
            pltpu.SemaphoreType.DMA,
        ],
        compiler_params=pltpu.CompilerParams(
            needs_layout_passes=False,
            disable_bounds_checks=True,
            disable_semaphore_checks=True,
        ),
    )
    def k(row_hbm, col_hbm, out_hbm, rowbuf, plane, sem, col_sem, row_sem):
        cid = lax.axis_index("c")
        sid = lax.axis_index("s")
        y = cid * 16 + sid

        cp_col = pltpu.async_copy(
            col_hbm.at[pl.ds(0, W)], plane.at[0, :, pl.ds(0, D)], col_sem
        )
        cp_row = pltpu.async_copy(row_hbm.at[pl.ds(y, 1)], rowbuf, row_sem)
        cp_row.wait()

        def _store_x(x, _):
            for j in range(D // _L):
                plane[0, x, pl.ds(D + j * _L, _L)] = rowbuf[0, pl.ds(j * _L, _L)]
            return 0

        lax.fori_loop(0, W, _store_x, 0)
        cp_col.wait()

        copies = [
            pltpu.async_copy(plane, out_hbm.at[b, pl.ds(y, 1)], sem)
            for b in range(B)
        ]
        for cp in copies:
            cp.wait()

    return k(row_embed, col_embed)


def kernel(x, row_embed, col_embed):
    B = x.shape[0]
    H, W = x.shape[-2], x.shape[-1]
    D = row_embed.shape[-1]
    out = _pos_embed_sc(row_embed, col_embed, B, H, W, D)
    return jnp.transpose(out, (0, 3, 1, 2))


# speedup vs baseline: 1.0070x; 1.0070x over previous
import functools

import jax
import jax.numpy as jnp
from jax import lax
from jax.experimental import pallas as pl
from jax.experimental.pallas import tpu as pltpu
from jax.experimental.pallas import tpu_sc as plsc

_L = 16


def _pos_embed_sc(row_embed, col_embed, B, H, W, D):
    C = 2 * D
    NW = 32
    assert H == NW and W <= col_embed.shape[0]
    NROW, DROW = row_embed.shape
    mesh = plsc.VectorSubcoreMesh(core_axis_name="c", subcore_axis_name="s")

    @functools.partial(
        pl.kernel,
        mesh=mesh,
        out_type=jax.ShapeDtypeStruct((B, H, W, C), jnp.float32),
        scratch_types=[
            pltpu.VMEM((1, DROW), jnp.float32),
            pltpu.VMEM((1, W, C), jnp.float32),
            pltpu.SemaphoreType.DMA,
            pltpu.SemaphoreType.DMA,
            pltpu.SemaphoreType.DMA,
        ],
    )
    def k(row_hbm, col_hbm, out_hbm, rowbuf, plane, sem, col_sem, row_sem):
        cid = lax.axis_index("c")
        sid = lax.axis_index("s")
        y = cid * 16 + sid

        cp_col = pltpu.async_copy(
            col_hbm.at[pl.ds(0, W)], plane.at[0, :, pl.ds(0, D)], col_sem
        )
        cp_row = pltpu.async_copy(row_hbm.at[pl.ds(y, 1)], rowbuf, row_sem)
        cp_row.wait()

        def _store_x(x, _):
            for j in range(D // _L):
                plane[0, x, pl.ds(D + j * _L, _L)] = rowbuf[0, pl.ds(j * _L, _L)]
            return 0

        lax.fori_loop(0, W, _store_x, 0)
        cp_col.wait()

        copies = [
            pltpu.async_copy(plane, out_hbm.at[b, pl.ds(y, 1)], sem)
            for b in range(B)
        ]
        for cp in copies:
            cp.wait()

    return k(row_embed, col_embed)


def kernel(x, row_embed, col_embed):
    B = x.shape[0]
    H, W = x.shape[-2], x.shape[-1]
    D = row_embed.shape[-1]
    out = _pos_embed_sc(row_embed, col_embed, B, H, W, D)
    return jnp.transpose(out, (0, 3, 1, 2))
